# baseline (device time: 125477 ns/iter reference)
import jax
import jax.numpy as jnp
from jax import lax
from jax.experimental import pallas as pl
from jax.experimental.pallas import tpu as pltpu

N_DEV = 32
LOG2 = 5
B, Sq, Hq, Dh = 2, 512, 8, 64
HD = Hq * Dh
SKV = 512
DMODEL = 768


def kernel(x, Wq, K_ext, V_ext, Wo):
    K2 = K_ext.reshape(B, SKV, HD)
    V2 = V_ext.reshape(B, SKV, HD)

    def body(x_ref, wq_ref, k_ref, v_ref, wo_ref, out_ref,
             o_acc, l_acc, o_send, l_send, o_recv, l_recv,
             o_ssem, o_rsem, l_ssem, l_rsem, exit_sem):
        my = lax.axis_index("i")

        qi = lax.broadcasted_iota(jnp.int32, (Sq, SKV), 0)
        ji = lax.broadcasted_iota(jnp.int32, (Sq, SKV), 1)
        mask = ((qi // 64) % 4) == ((ji // 64) % 4)

        wq = wq_ref[...].astype(jnp.bfloat16)
        for b in range(B):
            xb = x_ref[b, :, :].astype(jnp.bfloat16)
            qall = jax.lax.dot(xb, wq, preferred_element_type=jnp.float32)
            qall = qall.astype(jnp.bfloat16)
            for h in range(Hq):
                qh = qall[:, h * Dh:(h + 1) * Dh]
                kh = k_ref[b, :, h * Dh:(h + 1) * Dh].astype(jnp.bfloat16)
                s = jax.lax.dot_general(
                    qh, kh, (((1,), (1,)), ((), ())),
                    preferred_element_type=jnp.float32) * 0.125
                w = jnp.exp(jnp.where(mask, s, -1e9))
                l_acc[b, h, :] = jnp.sum(w, axis=1)
                wb = w.astype(jnp.bfloat16)
                vh = v_ref[b, :, h * Dh:(h + 1) * Dh].astype(jnp.bfloat16)
                o_acc[b, :, h * Dh:(h + 1) * Dh] = jax.lax.dot(
                    wb, vh, preferred_element_type=jnp.float32)

        for k in range(LOG2):
            partner = my ^ (1 << k)
            o_send[...] = o_acc[...].astype(jnp.bfloat16)
            l_send[...] = l_acc[...]
            ro = pltpu.make_async_remote_copy(
                src_ref=o_send, dst_ref=o_recv.at[k],
                send_sem=o_ssem.at[k], recv_sem=o_rsem.at[k],
                device_id=(partner,), device_id_type=pl.DeviceIdType.MESH)
            rl = pltpu.make_async_remote_copy(
                src_ref=l_send, dst_ref=l_recv.at[k],
                send_sem=l_ssem.at[k], recv_sem=l_rsem.at[k],
                device_id=(partner,), device_id_type=pl.DeviceIdType.MESH)
            ro.start()
            rl.start()
            ro.wait()
            rl.wait()
            o_acc[...] = o_acc[...] + o_recv[k, :, :, :].astype(jnp.float32)
            l_acc[...] = l_acc[...] + l_recv[k, :, :, :]

        wo = wo_ref[...].astype(jnp.bfloat16)
        for b in range(B):
            lb = l_acc[b, :, :]
            lfull = jnp.broadcast_to(
                lb.T[:, :, None], (Sq, Hq, Dh)).reshape(Sq, HD)
            ctx = (o_acc[b, :, :] / lfull).astype(jnp.bfloat16)
            out_ref[b, :, :] = jax.lax.dot(
                ctx, wo, preferred_element_type=jnp.float32)

        for k in range(LOG2):
            pl.semaphore_signal(
                exit_sem, inc=1,
                device_id=(my ^ (1 << k),),
                device_id_type=pl.DeviceIdType.MESH)
        pl.semaphore_wait(exit_sem, LOG2)

    return pl.pallas_call(
        body,
        out_shape=jax.ShapeDtypeStruct((B, Sq, DMODEL), jnp.float32),
        in_specs=[pl.BlockSpec(memory_space=pltpu.VMEM)] * 5,
        out_specs=pl.BlockSpec(memory_space=pltpu.VMEM),
        scratch_shapes=[
            pltpu.VMEM((B, Sq, HD), jnp.float32),
            pltpu.VMEM((B, Hq, Sq), jnp.float32),
            pltpu.VMEM((B, Sq, HD), jnp.bfloat16),
            pltpu.VMEM((B, Hq, Sq), jnp.float32),
            pltpu.VMEM((LOG2, B, Sq, HD), jnp.bfloat16),
            pltpu.VMEM((LOG2, B, Hq, Sq), jnp.float32),
            pltpu.SemaphoreType.DMA((LOG2,)),
            pltpu.SemaphoreType.DMA((LOG2,)),
            pltpu.SemaphoreType.DMA((LOG2,)),
            pltpu.SemaphoreType.DMA((LOG2,)),
            pltpu.SemaphoreType.REGULAR,
        ],
    )(x, Wq, K2, V2, Wo)


# device time: 76142 ns/iter; 1.6479x vs baseline; 1.6479x over previous
import jax
import jax.numpy as jnp
from jax import lax
from jax.experimental import pallas as pl
from jax.experimental.pallas import tpu as pltpu

N_DEV = 32
LOG2 = 5
B, Sq, Hq, Dh = 2, 512, 8, 64
HD = Hq * Dh
SKV = 512
DMODEL = 768
OWN = Sq // N_DEV

SIZES = [Sq >> (k + 1) for k in range(LOG2)]


def kernel(x, Wq, K_ext, V_ext, Wo):
    K2 = K_ext.reshape(B, SKV, HD)
    V2 = V_ext.reshape(B, SKV, HD)

    def body(x_ref, wq_ref, k_ref, v_ref, wo_ref, out_ref,
             o_acc, l_acc, o_send, l_send,
             or0, or1, or2, or3, or4, lr0, lr1, lr2, lr3, lr4,
             rs_ssem, rs_rsem, l_ssem, l_rsem, ag_ssem, ag_rsem,
             o_final, lt_buf, exit_sem):
        my = lax.axis_index("i")
        o_recv = [or0, or1, or2, or3, or4]
        l_recv = [lr0, lr1, lr2, lr3, lr4]

        qi = lax.broadcasted_iota(jnp.int32, (Sq, SKV), 0)
        ji = lax.broadcasted_iota(jnp.int32, (Sq, SKV), 1)
        mask = ((qi // 64) % 4) == ((ji // 64) % 4)

        wq = wq_ref[...].astype(jnp.bfloat16)
        for b in range(B):
            xb = x_ref[b, :, :].astype(jnp.bfloat16)
            qall = jax.lax.dot(xb, wq, preferred_element_type=jnp.float32)
            qall = qall.astype(jnp.bfloat16)
            for h in range(Hq):
                qh = qall[:, h * Dh:(h + 1) * Dh]
                kh = k_ref[b, :, h * Dh:(h + 1) * Dh].astype(jnp.bfloat16)
                s = jax.lax.dot_general(
                    qh, kh, (((1,), (1,)), ((), ())),
                    preferred_element_type=jnp.float32) * 0.125
                w = jnp.exp(jnp.where(mask, s, -1e9))
                l_acc[b, h, :] = jnp.sum(w, axis=1)
                wb = w.astype(jnp.bfloat16)
                vh = v_ref[b, :, h * Dh:(h + 1) * Dh].astype(jnp.bfloat16)
                o_acc[b, :, h * Dh:(h + 1) * Dh] = jax.lax.dot(
                    wb, vh, preferred_element_type=jnp.float32)

        lo = jnp.int32(0)
        for k in range(LOG2):
            sz = SIZES[k]
            bit = (my >> k) & 1
            partner = my ^ (1 << k)
            keep_lo = lo + bit * sz
            send_lo = lo + (1 - bit) * sz
            o_send[:, :sz, :] = o_acc[
                :, pl.ds(send_lo, sz), :].astype(jnp.bfloat16)
            l_send[...] = l_acc[...]
            ro = pltpu.make_async_remote_copy(
                src_ref=o_send.at[:, :sz, :], dst_ref=o_recv[k],
                send_sem=rs_ssem.at[k], recv_sem=rs_rsem.at[k],
                device_id=(partner,), device_id_type=pl.DeviceIdType.MESH)
            rl = pltpu.make_async_remote_copy(
                src_ref=l_send, dst_ref=l_recv[k],
                send_sem=l_ssem.at[k], recv_sem=l_rsem.at[k],
                device_id=(partner,), device_id_type=pl.DeviceIdType.MESH)
            ro.start()
            rl.start()
            ro.wait()
            rl.wait()
            o_acc[:, pl.ds(keep_lo, sz), :] = (
                o_acc[:, pl.ds(keep_lo, sz), :]
                + o_recv[k][:, :, :].astype(jnp.float32))
            l_acc[...] = l_acc[...] + l_recv[k][:, :, :]
            lo = keep_lo

        for b in range(B):
            lt_buf[b, :, :] = l_acc[b, :, :].T
            lb = lt_buf[b, pl.ds(lo, OWN), :]
            lfull = jnp.broadcast_to(
                lb[:, :, None], (OWN, Hq, Dh)).reshape(OWN, HD)
            o_final[b, pl.ds(lo, OWN), :] = (
                o_acc[b, pl.ds(lo, OWN), :] / lfull).astype(jnp.bfloat16)

        r_lo = lo
        for k in reversed(range(LOG2)):
            sz = SIZES[k]
            partner = my ^ (1 << k)
            ag = pltpu.make_async_remote_copy(
                src_ref=o_final.at[:, pl.ds(r_lo, sz), :],
                dst_ref=o_final.at[:, pl.ds(r_lo, sz), :],
                send_sem=ag_ssem.at[k], recv_sem=ag_rsem.at[k],
                device_id=(partner,), device_id_type=pl.DeviceIdType.MESH)
            ag.start()
            ag.wait()
            r_lo = r_lo - ((my >> k) & 1) * sz

        wo = wo_ref[...].astype(jnp.bfloat16)
        for b in range(B):
            out_ref[b, :, :] = jax.lax.dot(
                o_final[b, :, :], wo, preferred_element_type=jnp.float32)

        for k in range(LOG2):
            pl.semaphore_signal(
                exit_sem, inc=1,
                device_id=(my ^ (1 << k),),
                device_id_type=pl.DeviceIdType.MESH)
        pl.semaphore_wait(exit_sem, LOG2)

    return pl.pallas_call(
        body,
        out_shape=jax.ShapeDtypeStruct((B, Sq, DMODEL), jnp.float32),
        in_specs=[pl.BlockSpec(memory_space=pltpu.VMEM)] * 5,
        out_specs=pl.BlockSpec(memory_space=pltpu.VMEM),
        scratch_shapes=[
            pltpu.VMEM((B, Sq, HD), jnp.float32),
            pltpu.VMEM((B, Hq, Sq), jnp.float32),
            pltpu.VMEM((B, SIZES[0], HD), jnp.bfloat16),
            pltpu.VMEM((B, Hq, Sq), jnp.float32),
            *[pltpu.VMEM((B, s, HD), jnp.bfloat16) for s in SIZES],
            *[pltpu.VMEM((B, Hq, Sq), jnp.float32) for _ in SIZES],
            pltpu.SemaphoreType.DMA((LOG2,)),
            pltpu.SemaphoreType.DMA((LOG2,)),
            pltpu.SemaphoreType.DMA((LOG2,)),
            pltpu.SemaphoreType.DMA((LOG2,)),
            pltpu.SemaphoreType.DMA((LOG2,)),
            pltpu.SemaphoreType.DMA((LOG2,)),
            pltpu.VMEM((B, Sq, HD), jnp.bfloat16),
            pltpu.VMEM((B, Sq, Hq), jnp.float32),
            pltpu.SemaphoreType.REGULAR,
        ],
    )(x, Wq, K2, V2, Wo)
